# baseline (speedup 1.0000x reference)
import functools

import jax
import jax.numpy as jnp
from jax import lax
from jax.experimental import pallas as pl
from jax.experimental.pallas import tpu as pltpu
from jax.experimental.pallas import tpu_sc as plsc

N = 10000
E = 320000
D = 128
L = 128
C = 40

NC = 2
NS = 16
NW = NC * NS
K = 128
CPW = 80
EPAD = NW * CPW * K
SEGC = 16
NSIO = 10
RPS = N // NSIO
ZROWS = 40


def _hop_body(x_hbm, w_hbm, recv_hbm, send_hbm, out_hbm,
              ridx_v, sidx_v, w_all, rows0, rows1, zb_v, acc_sh,
              isem, gsem0, gsem1):
    cid = lax.axis_index("c")
    sid = lax.axis_index("s")
    wid = cid * NS + sid
    r0 = wid * CPW

    zvec = jnp.zeros((16,), jnp.float32)

    @pl.when(sid < NSIO)
    def _zero():
        @pl.loop(0, ZROWS * (D // 16))
        def _zero_fill(i):
            r = i // (D // 16)
            c = i % (D // 16)
            zb_v[r, pl.ds(pl.multiple_of(c * 16, 16), 16)] = zvec

        @pl.loop(0, RPS // ZROWS)
        def _zero_acc(j):
            pltpu.sync_copy(zb_v,
                            acc_sh.at[pl.ds(sid * RPS + j * ZROWS, ZROWS)])

    plsc.subcore_barrier()

    rows = (rows0, rows1)
    gsems = (gsem0, gsem1)

    @pl.loop(0, CPW // SEGC)
    def _segment(s):
        sbase = r0 + s * SEGC
        c_r = pltpu.async_copy(recv_hbm.at[pl.ds(sbase, SEGC)], ridx_v, isem)
        c_s = pltpu.async_copy(send_hbm.at[pl.ds(sbase, SEGC)], sidx_v, isem)
        c_w = pltpu.async_copy(w_hbm.at[pl.ds(sbase, SEGC)], w_all, isem)
        c_r.wait()
        c_s.wait()
        c_w.wait()

        pltpu.async_copy(x_hbm.at[ridx_v.at[0]], rows0, gsem0)
        pltpu.async_copy(x_hbm.at[ridx_v.at[1]], rows1, gsem1)

        @pl.loop(0, SEGC, step=2)
        def _chunk(i):
            for b in range(2):
                ic = i + b
                rows_b = rows[b]
                pltpu.make_async_copy(x_hbm.at[ridx_v.at[ic]], rows_b,
                                      gsems[b]).wait()

                @pl.loop(0, K // 16)
                def _scale(g):
                    w16 = w_all[ic, pl.ds(pl.multiple_of(g * 16, 16), 16)]
                    for t in range(16):
                        wk = w16[t]
                        e = g * 16 + t
                        for j in range(D // 16):
                            sl = pl.ds(j * 16, 16)
                            rows_b[e, sl] = rows_b[e, sl] * wk

                pltpu.sync_copy(rows_b, acc_sh.at[sidx_v.at[ic]], add=True)

                @pl.when(ic + 2 < SEGC)
                def _prefetch():
                    pltpu.async_copy(x_hbm.at[ridx_v.at[ic + 2]], rows_b,
                                     gsems[b])

    plsc.subcore_barrier()

    @pl.when(sid < NSIO)
    def _copy_out():
        pltpu.sync_copy(acc_sh.at[pl.ds(sid * RPS, RPS)],
                        out_hbm.at[cid, pl.ds(sid * RPS, RPS)])


_hop = functools.partial(
    pl.kernel,
    out_type=jax.ShapeDtypeStruct((NC, N, L), jnp.float32),
    mesh=plsc.VectorSubcoreMesh(core_axis_name="c", subcore_axis_name="s",
                                num_cores=NC, num_subcores=NS),
    scratch_types=[
        pltpu.VMEM((SEGC, K), jnp.int32),
        pltpu.VMEM((SEGC, K), jnp.int32),
        pltpu.VMEM((SEGC, K), jnp.float32),
        pltpu.VMEM((K, L), jnp.float32),
        pltpu.VMEM((K, L), jnp.float32),
        pltpu.VMEM((ZROWS, L), jnp.float32),
        pltpu.VMEM_SHARED((N, L), jnp.float32),
        pltpu.SemaphoreType.DMA,
        pltpu.SemaphoreType.DMA,
        pltpu.SemaphoreType.DMA,
    ],
)(_hop_body)


BM = 2000


def _encoder_body(x_ref, w_ref, b_ref, o_ref):
    y = jnp.dot(x_ref[...], w_ref[...], preferred_element_type=jnp.float32)
    o_ref[...] = jnp.maximum(y + b_ref[...], 0.0)


def _encoder(x, w, b):
    return pl.pallas_call(
        _encoder_body,
        grid=(N // BM,),
        in_specs=[
            pl.BlockSpec((BM, D), lambda i: (i, 0)),
            pl.BlockSpec((D, L), lambda i: (0, 0)),
            pl.BlockSpec((1, L), lambda i: (0, 0)),
        ],
        out_specs=pl.BlockSpec((BM, L), lambda i: (i, 0)),
        out_shape=jax.ShapeDtypeStruct((N, L), jnp.float32),
    )(x, w, b.reshape(1, L))


def _update_body(p_ref, w_ref, b_ref, o_ref):
    conv = p_ref[0] + p_ref[1]
    h = jnp.dot(conv, w_ref[...], preferred_element_type=jnp.float32)
    o_ref[...] = jnp.maximum(h + b_ref[...], 0.0) + conv


def _update(parts, w, b):
    return pl.pallas_call(
        _update_body,
        grid=(N // BM,),
        in_specs=[
            pl.BlockSpec((NC, BM, L), lambda i: (0, i, 0)),
            pl.BlockSpec((L, L), lambda i: (0, 0)),
            pl.BlockSpec((1, L), lambda i: (0, 0)),
        ],
        out_specs=pl.BlockSpec((BM, L), lambda i: (i, 0)),
        out_shape=jax.ShapeDtypeStruct((N, L), jnp.float32),
    )(parts, w, b.reshape(1, L))


def _update_dec_body(p_ref, w_ref, b_ref, dw_ref, db_ref, o_ref):
    conv = p_ref[0] + p_ref[1]
    h = jnp.dot(conv, w_ref[...], preferred_element_type=jnp.float32)
    x = jnp.maximum(h + b_ref[...], 0.0) + conv
    o_ref[...] = jnp.dot(x, dw_ref[...],
                         preferred_element_type=jnp.float32) + db_ref[...]


def _update_dec(parts, w, b, dw, db):
    return pl.pallas_call(
        _update_dec_body,
        grid=(N // BM,),
        in_specs=[
            pl.BlockSpec((NC, BM, L), lambda i: (0, i, 0)),
            pl.BlockSpec((L, L), lambda i: (0, 0)),
            pl.BlockSpec((1, L), lambda i: (0, 0)),
            pl.BlockSpec((L, L), lambda i: (0, 0)),
            pl.BlockSpec((1, L), lambda i: (0, 0)),
        ],
        out_specs=pl.BlockSpec((BM, L), lambda i: (i, 0)),
        out_shape=jax.ShapeDtypeStruct((N, L), jnp.float32),
    )(parts, w, b.reshape(1, L), dw, db.reshape(1, L))


def kernel(nodes, edges, senders, receivers, enc_W, enc_b, core0_W, core0_b,
           core1_W, core1_b, dec_W, dec_b):
    w = edges.reshape(E)
    senders = senders.astype(jnp.int32)
    receivers = receivers.astype(jnp.int32)

    npad = EPAD - E
    w_p = jnp.concatenate([w, jnp.zeros((npad,), jnp.float32)]
                          ).reshape(EPAD // K, K)
    recv_p = jnp.concatenate([receivers, jnp.zeros((npad,), jnp.int32)]
                             ).reshape(EPAD // K, K)
    send_p = jnp.concatenate([senders, jnp.zeros((npad,), jnp.int32)]
                             ).reshape(EPAD // K, K)

    x = _encoder(nodes, enc_W, enc_b)
    parts = _hop(x, w_p, recv_p, send_p)
    x = _update(parts, core0_W, core0_b)
    parts = _hop(x, w_p, recv_p, send_p)

    dw_pad = jnp.zeros((L, L), jnp.float32).at[:, :C].set(dec_W)
    db_pad = jnp.zeros((L,), jnp.float32).at[:C].set(dec_b)
    out = _update_dec(parts, core1_W, core1_b, dw_pad, db_pad)
    return out[:, :C]


# speedup vs baseline: 1.0012x; 1.0012x over previous
import functools

import jax
import jax.numpy as jnp
from jax import lax
from jax.experimental import pallas as pl
from jax.experimental.pallas import tpu as pltpu
from jax.experimental.pallas import tpu_sc as plsc

N = 10000
E = 320000
D = 128
L = 128
C = 40

NC = 2
NS = 16
NW = NC * NS
K = 128
CPW = 80
EPAD = NW * CPW * K
SEGC = 16
NSIO = 10
RPS = N // NSIO
ZROWS = 40


def _hop_body(x_hbm, w_hbm, recv_hbm, send_hbm, out_hbm,
              ridx_v, sidx_v, w_all, rows0, rows1, zb_v, acc_sh,
              isem, gsem0, gsem1):
    cid = lax.axis_index("c")
    sid = lax.axis_index("s")
    wid = cid * NS + sid
    r0 = wid * CPW

    zvec = jnp.zeros((16,), jnp.float32)

    @pl.when(sid < NSIO)
    def _zero():
        @pl.loop(0, ZROWS * (D // 16))
        def _zero_fill(i):
            r = i // (D // 16)
            c = i % (D // 16)
            zb_v[r, pl.ds(pl.multiple_of(c * 16, 16), 16)] = zvec

        @pl.loop(0, RPS // ZROWS)
        def _zero_acc(j):
            pltpu.sync_copy(zb_v,
                            acc_sh.at[pl.ds(sid * RPS + j * ZROWS, ZROWS)])

    plsc.subcore_barrier()

    rows = (rows0, rows1)
    gsems = (gsem0, gsem1)

    @pl.loop(0, CPW // SEGC)
    def _segment(s):
        sbase = r0 + s * SEGC
        c_r = pltpu.async_copy(recv_hbm.at[pl.ds(sbase, SEGC)], ridx_v, isem)
        c_s = pltpu.async_copy(send_hbm.at[pl.ds(sbase, SEGC)], sidx_v, isem)
        c_w = pltpu.async_copy(w_hbm.at[pl.ds(sbase, SEGC)], w_all, isem)
        c_r.wait()
        c_s.wait()
        c_w.wait()

        for b in range(2):
            for q in range(4):
                pltpu.async_copy(
                    x_hbm.at[ridx_v.at[b, pl.ds(q * 32, 32)]],
                    rows[b].at[pl.ds(q * 32, 32)], gsems[b])

        @pl.loop(0, SEGC, step=2)
        def _chunk(i):
            for b in range(2):
                ic = i + b
                rows_b = rows[b]
                for q in range(4):
                    pltpu.make_async_copy(
                        x_hbm.at[ridx_v.at[ic, pl.ds(q * 32, 32)]],
                        rows_b.at[pl.ds(q * 32, 32)], gsems[b]).wait()

                @pl.loop(0, K // 16)
                def _scale(g):
                    w16 = w_all[ic, pl.ds(pl.multiple_of(g * 16, 16), 16)]
                    for t in range(16):
                        wk = w16[t]
                        e = g * 16 + t
                        for j in range(D // 16):
                            sl = pl.ds(j * 16, 16)
                            rows_b[e, sl] = rows_b[e, sl] * wk

                pltpu.sync_copy(rows_b, acc_sh.at[sidx_v.at[ic]], add=True)

                @pl.when(ic + 2 < SEGC)
                def _prefetch():
                    for q in range(4):
                        pltpu.async_copy(
                            x_hbm.at[ridx_v.at[ic + 2, pl.ds(q * 32, 32)]],
                            rows_b.at[pl.ds(q * 32, 32)], gsems[b])

    plsc.subcore_barrier()

    @pl.when(sid < NSIO)
    def _copy_out():
        pltpu.sync_copy(acc_sh.at[pl.ds(sid * RPS, RPS)],
                        out_hbm.at[cid, pl.ds(sid * RPS, RPS)])


_hop = functools.partial(
    pl.kernel,
    out_type=jax.ShapeDtypeStruct((NC, N, L), jnp.float32),
    mesh=plsc.VectorSubcoreMesh(core_axis_name="c", subcore_axis_name="s",
                                num_cores=NC, num_subcores=NS),
    scratch_types=[
        pltpu.VMEM((SEGC, K), jnp.int32),
        pltpu.VMEM((SEGC, K), jnp.int32),
        pltpu.VMEM((SEGC, K), jnp.float32),
        pltpu.VMEM((K, L), jnp.float32),
        pltpu.VMEM((K, L), jnp.float32),
        pltpu.VMEM((ZROWS, L), jnp.float32),
        pltpu.VMEM_SHARED((N, L), jnp.float32),
        pltpu.SemaphoreType.DMA,
        pltpu.SemaphoreType.DMA,
        pltpu.SemaphoreType.DMA,
    ],
)(_hop_body)


BM = 2000


def _encoder_body(x_ref, w_ref, b_ref, o_ref):
    y = jnp.dot(x_ref[...], w_ref[...], preferred_element_type=jnp.float32)
    o_ref[...] = jnp.maximum(y + b_ref[...], 0.0)


def _encoder(x, w, b):
    return pl.pallas_call(
        _encoder_body,
        grid=(N // BM,),
        in_specs=[
            pl.BlockSpec((BM, D), lambda i: (i, 0)),
            pl.BlockSpec((D, L), lambda i: (0, 0)),
            pl.BlockSpec((1, L), lambda i: (0, 0)),
        ],
        out_specs=pl.BlockSpec((BM, L), lambda i: (i, 0)),
        out_shape=jax.ShapeDtypeStruct((N, L), jnp.float32),
    )(x, w, b.reshape(1, L))


def _update_body(p_ref, w_ref, b_ref, o_ref):
    conv = p_ref[0] + p_ref[1]
    h = jnp.dot(conv, w_ref[...], preferred_element_type=jnp.float32)
    o_ref[...] = jnp.maximum(h + b_ref[...], 0.0) + conv


def _update(parts, w, b):
    return pl.pallas_call(
        _update_body,
        grid=(N // BM,),
        in_specs=[
            pl.BlockSpec((NC, BM, L), lambda i: (0, i, 0)),
            pl.BlockSpec((L, L), lambda i: (0, 0)),
            pl.BlockSpec((1, L), lambda i: (0, 0)),
        ],
        out_specs=pl.BlockSpec((BM, L), lambda i: (i, 0)),
        out_shape=jax.ShapeDtypeStruct((N, L), jnp.float32),
    )(parts, w, b.reshape(1, L))


def _update_dec_body(p_ref, w_ref, b_ref, dw_ref, db_ref, o_ref):
    conv = p_ref[0] + p_ref[1]
    h = jnp.dot(conv, w_ref[...], preferred_element_type=jnp.float32)
    x = jnp.maximum(h + b_ref[...], 0.0) + conv
    o_ref[...] = jnp.dot(x, dw_ref[...],
                         preferred_element_type=jnp.float32) + db_ref[...]


def _update_dec(parts, w, b, dw, db):
    return pl.pallas_call(
        _update_dec_body,
        grid=(N // BM,),
        in_specs=[
            pl.BlockSpec((NC, BM, L), lambda i: (0, i, 0)),
            pl.BlockSpec((L, L), lambda i: (0, 0)),
            pl.BlockSpec((1, L), lambda i: (0, 0)),
            pl.BlockSpec((L, L), lambda i: (0, 0)),
            pl.BlockSpec((1, L), lambda i: (0, 0)),
        ],
        out_specs=pl.BlockSpec((BM, L), lambda i: (i, 0)),
        out_shape=jax.ShapeDtypeStruct((N, L), jnp.float32),
    )(parts, w, b.reshape(1, L), dw, db.reshape(1, L))


def kernel(nodes, edges, senders, receivers, enc_W, enc_b, core0_W, core0_b,
           core1_W, core1_b, dec_W, dec_b):
    w = edges.reshape(E)
    senders = senders.astype(jnp.int32)
    receivers = receivers.astype(jnp.int32)

    npad = EPAD - E
    w_p = jnp.concatenate([w, jnp.zeros((npad,), jnp.float32)]
                          ).reshape(EPAD // K, K)
    recv_p = jnp.concatenate([receivers, jnp.zeros((npad,), jnp.int32)]
                             ).reshape(EPAD // K, K)
    send_p = jnp.concatenate([senders, jnp.zeros((npad,), jnp.int32)]
                             ).reshape(EPAD // K, K)

    x = _encoder(nodes, enc_W, enc_b)
    parts = _hop(x, w_p, recv_p, send_p)
    x = _update(parts, core0_W, core0_b)
    parts = _hop(x, w_p, recv_p, send_p)

    dw_pad = jnp.zeros((L, L), jnp.float32).at[:, :C].set(dec_W)
    db_pad = jnp.zeros((L,), jnp.float32).at[:C].set(dec_b)
    out = _update_dec(parts, core1_W, core1_b, dw_pad, db_pad)
    return out[:, :C]


# speedup vs baseline: 1.1359x; 1.1346x over previous
import functools

import jax
import jax.numpy as jnp
from jax import lax
from jax.experimental import pallas as pl
from jax.experimental.pallas import tpu as pltpu
from jax.experimental.pallas import tpu_sc as plsc

N = 10000
E = 320000
D = 128
L = 128
C = 40
H = L // 2

NC = 2
NS = 16
K = 128
CPS = 160
EPAD = NS * CPS * K
SEGC = 16
NSIO = 10
RPS = N // NSIO
ZROWS = 40


def _hop_body(xh0_hbm, xh1_hbm, w_hbm, recv_hbm, send_hbm, out_hbm,
              ridx_v, sidx_v, w_all, rows0, rows1, zb_v, x_sh, acc_sh,
              isem, gsem0, gsem1):
    cid = lax.axis_index("c")
    sid = lax.axis_index("s")
    r0 = sid * CPS

    zvec = jnp.zeros((16,), jnp.float32)

    @pl.when((sid < NSIO) & (cid == 0))
    def _stage0():
        pltpu.sync_copy(xh0_hbm.at[pl.ds(sid * RPS, RPS)],
                        x_sh.at[pl.ds(sid * RPS, RPS)])

    @pl.when((sid < NSIO) & (cid == 1))
    def _stage1():
        pltpu.sync_copy(xh1_hbm.at[pl.ds(sid * RPS, RPS)],
                        x_sh.at[pl.ds(sid * RPS, RPS)])

    @pl.when(sid < NSIO)
    def _zero():
        @pl.loop(0, ZROWS * (H // 16))
        def _zero_fill(i):
            r = i // (H // 16)
            c = i % (H // 16)
            zb_v[r, pl.ds(pl.multiple_of(c * 16, 16), 16)] = zvec

        @pl.loop(0, RPS // ZROWS)
        def _zero_acc(j):
            pltpu.sync_copy(zb_v,
                            acc_sh.at[pl.ds(sid * RPS + j * ZROWS, ZROWS)])

    plsc.subcore_barrier()

    rows = (rows0, rows1)
    gsems = (gsem0, gsem1)

    @pl.loop(0, CPS // SEGC)
    def _segment(s):
        sbase = r0 + s * SEGC
        c_r = pltpu.async_copy(recv_hbm.at[pl.ds(sbase, SEGC)], ridx_v, isem)
        c_s = pltpu.async_copy(send_hbm.at[pl.ds(sbase, SEGC)], sidx_v, isem)
        c_w = pltpu.async_copy(w_hbm.at[pl.ds(sbase, SEGC)], w_all, isem)
        c_r.wait()
        c_s.wait()
        c_w.wait()

        for b in range(2):
            pltpu.async_copy(x_sh.at[ridx_v.at[b]], rows[b], gsems[b])

        @pl.loop(0, SEGC, step=2)
        def _chunk(i):
            for b in range(2):
                ic = i + b
                rows_b = rows[b]
                pltpu.make_async_copy(x_sh.at[ridx_v.at[ic]], rows_b,
                                      gsems[b]).wait()

                @pl.loop(0, K // 16)
                def _scale(g):
                    w16 = w_all[ic, pl.ds(pl.multiple_of(g * 16, 16), 16)]
                    for t in range(16):
                        wk = w16[t]
                        e = g * 16 + t
                        for j in range(H // 16):
                            sl = pl.ds(j * 16, 16)
                            rows_b[e, sl] = rows_b[e, sl] * wk

                pltpu.sync_copy(rows_b, acc_sh.at[sidx_v.at[ic]], add=True)

                @pl.when(ic + 2 < SEGC)
                def _prefetch():
                    pltpu.async_copy(x_sh.at[ridx_v.at[ic + 2]], rows_b,
                                     gsems[b])

    plsc.subcore_barrier()

    @pl.when(sid < NSIO)
    def _copy_out():
        pltpu.sync_copy(acc_sh.at[pl.ds(sid * RPS, RPS)],
                        out_hbm.at[cid, pl.ds(sid * RPS, RPS)])


_hop = functools.partial(
    pl.kernel,
    out_type=jax.ShapeDtypeStruct((NC, N, H), jnp.float32),
    mesh=plsc.VectorSubcoreMesh(core_axis_name="c", subcore_axis_name="s",
                                num_cores=NC, num_subcores=NS),
    compiler_params=pltpu.CompilerParams(use_tc_tiling_on_sc=False),
    scratch_types=[
        pltpu.VMEM((SEGC, K), jnp.int32),
        pltpu.VMEM((SEGC, K), jnp.int32),
        pltpu.VMEM((SEGC, K), jnp.float32),
        pltpu.VMEM((K, H), jnp.float32),
        pltpu.VMEM((K, H), jnp.float32),
        pltpu.VMEM((ZROWS, H), jnp.float32),
        pltpu.VMEM_SHARED((N, H), jnp.float32),
        pltpu.VMEM_SHARED((N, H), jnp.float32),
        pltpu.SemaphoreType.DMA,
        pltpu.SemaphoreType.DMA,
        pltpu.SemaphoreType.DMA,
    ],
)(_hop_body)


BM = 2000


def _encoder_body(x_ref, w_ref, b_ref, o0_ref, o1_ref):
    y = jnp.dot(x_ref[...], w_ref[...], preferred_element_type=jnp.float32)
    y = jnp.maximum(y + b_ref[...], 0.0)
    o0_ref[...] = y[:, :H]
    o1_ref[...] = y[:, H:]


def _encoder(x, w, b):
    return pl.pallas_call(
        _encoder_body,
        grid=(N // BM,),
        in_specs=[
            pl.BlockSpec((BM, D), lambda i: (i, 0)),
            pl.BlockSpec((D, L), lambda i: (0, 0)),
            pl.BlockSpec((1, L), lambda i: (0, 0)),
        ],
        out_specs=[pl.BlockSpec((BM, H), lambda i: (i, 0)),
                   pl.BlockSpec((BM, H), lambda i: (i, 0))],
        out_shape=[jax.ShapeDtypeStruct((N, H), jnp.float32),
                   jax.ShapeDtypeStruct((N, H), jnp.float32)],
    )(x, w, b.reshape(1, L))


def _update_body(p_ref, w_ref, b_ref, o0_ref, o1_ref):
    conv = jnp.concatenate([p_ref[0], p_ref[1]], axis=1)
    h = jnp.dot(conv, w_ref[...], preferred_element_type=jnp.float32)
    x = jnp.maximum(h + b_ref[...], 0.0) + conv
    o0_ref[...] = x[:, :H]
    o1_ref[...] = x[:, H:]


def _update(parts, w, b):
    return pl.pallas_call(
        _update_body,
        grid=(N // BM,),
        in_specs=[
            pl.BlockSpec((NC, BM, H), lambda i: (0, i, 0)),
            pl.BlockSpec((L, L), lambda i: (0, 0)),
            pl.BlockSpec((1, L), lambda i: (0, 0)),
        ],
        out_specs=[pl.BlockSpec((BM, H), lambda i: (i, 0)),
                   pl.BlockSpec((BM, H), lambda i: (i, 0))],
        out_shape=[jax.ShapeDtypeStruct((N, H), jnp.float32),
                   jax.ShapeDtypeStruct((N, H), jnp.float32)],
    )(parts, w, b.reshape(1, L))


def _update_dec_body(p_ref, w_ref, b_ref, dw_ref, db_ref, o_ref):
    conv = jnp.concatenate([p_ref[0], p_ref[1]], axis=1)
    h = jnp.dot(conv, w_ref[...], preferred_element_type=jnp.float32)
    x = jnp.maximum(h + b_ref[...], 0.0) + conv
    o_ref[...] = jnp.dot(x, dw_ref[...],
                         preferred_element_type=jnp.float32) + db_ref[...]


def _update_dec(parts, w, b, dw, db):
    return pl.pallas_call(
        _update_dec_body,
        grid=(N // BM,),
        in_specs=[
            pl.BlockSpec((NC, BM, H), lambda i: (0, i, 0)),
            pl.BlockSpec((L, L), lambda i: (0, 0)),
            pl.BlockSpec((1, L), lambda i: (0, 0)),
            pl.BlockSpec((L, L), lambda i: (0, 0)),
            pl.BlockSpec((1, L), lambda i: (0, 0)),
        ],
        out_specs=pl.BlockSpec((BM, L), lambda i: (i, 0)),
        out_shape=jax.ShapeDtypeStruct((N, L), jnp.float32),
    )(parts, w, b.reshape(1, L), dw, db.reshape(1, L))


def kernel(nodes, edges, senders, receivers, enc_W, enc_b, core0_W, core0_b,
           core1_W, core1_b, dec_W, dec_b):
    w = edges.reshape(E)
    senders = senders.astype(jnp.int32)
    receivers = receivers.astype(jnp.int32)

    npad = EPAD - E
    w_p = jnp.concatenate([w, jnp.zeros((npad,), jnp.float32)]
                          ).reshape(EPAD // K, K)
    recv_p = jnp.concatenate([receivers, jnp.zeros((npad,), jnp.int32)]
                             ).reshape(EPAD // K, K)
    send_p = jnp.concatenate([senders, jnp.zeros((npad,), jnp.int32)]
                             ).reshape(EPAD // K, K)

    xh0, xh1 = _encoder(nodes, enc_W, enc_b)
    parts = _hop(xh0, xh1, w_p, recv_p, send_p)
    xh0, xh1 = _update(parts, core0_W, core0_b)
    parts = _hop(xh0, xh1, w_p, recv_p, send_p)

    dw_pad = jnp.zeros((L, L), jnp.float32).at[:, :C].set(dec_W)
    db_pad = jnp.zeros((L,), jnp.float32).at[:C].set(dec_b)
    out = _update_dec(parts, core1_W, core1_b, dw_pad, db_pad)
    return out[:, :C]
